# baseline (speedup 1.0000x reference)
import functools
import jax
import jax.numpy as jnp
from jax import lax
from jax.experimental import pallas as pl
from jax.experimental.pallas import tpu as pltpu
from jax.experimental.pallas import tpu_sc as plsc

_TB = 2048
_NE = 64
_NC = 2
_NS = 16
_NW = _NC * _NS
_L = 16
_UNROLL = 4
_NSLICE = 2


def _logits_body(x_ref, w_ref, b_ref, out_ref):
    x = x_ref[...]
    w = w_ref[...]
    lt = jax.lax.dot_general(
        w, x, (((1,), (1,)), ((), ())), preferred_element_type=jnp.float32
    )
    out_ref[...] = lt + b_ref[...]


def _tc_logits_t(hidden_states, gate_w, gate_b):
    T, H = hidden_states.shape
    return pl.pallas_call(
        _logits_body,
        grid=(T // _TB,),
        in_specs=[
            pl.BlockSpec((_TB, H), lambda i: (i, 0)),
            pl.BlockSpec((_NE, H), lambda i: (0, 0)),
            pl.BlockSpec((_NE, 1), lambda i: (0, 0)),
        ],
        out_specs=pl.BlockSpec((_NE, _TB), lambda i: (0, i)),
        out_shape=jax.ShapeDtypeStruct((_NE, T), jnp.float32),
    )(hidden_states, gate_w, gate_b.reshape(_NE, 1))


def _make_sc_router(T):
    tpw = T // _NW
    mesh = plsc.VectorSubcoreMesh(core_axis_name="c", subcore_axis_name="s")

    @functools.partial(
        pl.kernel,
        mesh=mesh,
        out_type=[
            jax.ShapeDtypeStruct((T,), jnp.float32),
            jax.ShapeDtypeStruct((T,), jnp.float32),
            jax.ShapeDtypeStruct((T,), jnp.int32),
            jax.ShapeDtypeStruct((T,), jnp.int32),
        ],
        scratch_types=[
            pltpu.VMEM((_NE, tpw), jnp.float32),
            pltpu.VMEM((tpw,), jnp.float32),
            pltpu.VMEM((tpw,), jnp.float32),
            pltpu.VMEM((tpw,), jnp.int32),
            pltpu.VMEM((tpw,), jnp.int32),
        ],
    )
    def sc_router(logits_hbm, w1_hbm, w2_hbm, i1_hbm, i2_hbm,
                  chunk, w1v, w2v, i1v, i2v):
        wid = lax.axis_index("s") * _NC + lax.axis_index("c")
        base = wid * tpw
        pltpu.sync_copy(logits_hbm.at[:, pl.ds(base, tpw)], chunk)

        def group(g, _):
            g16 = g * _L
            neg = jnp.full((_L,), -jnp.inf, jnp.float32)
            zz = jnp.zeros((_L,), jnp.int32)

            def estep(k, c):
                for d in range(_UNROLL):
                    m1, m2, j1, j2 = c
                    e = k * _UNROLL + d
                    v = chunk[e, pl.ds(g16, _L)]
                    ev = jnp.full((_L,), e, jnp.int32)
                    gt1 = v > m1
                    gt2 = v > m2
                    m2n = jnp.where(gt1, m1, jnp.where(gt2, v, m2))
                    j2n = jnp.where(gt1, j1, jnp.where(gt2, ev, j2))
                    m1n = jnp.where(gt1, v, m1)
                    j1n = jnp.where(gt1, ev, j1)
                    c = (m1n, m2n, j1n, j2n)
                return c

            m1, m2, j1, j2 = lax.fori_loop(
                0, _NE // _UNROLL, estep, (neg, neg, zz, zz)
            )
            ex = jnp.exp(m2 - m1)
            wa = 1.0 / (1.0 + ex)
            w1v[pl.ds(g16, _L)] = wa
            w2v[pl.ds(g16, _L)] = ex * wa
            i1v[pl.ds(g16, _L)] = j1
            i2v[pl.ds(g16, _L)] = j2
            return 0

        lax.fori_loop(0, tpw // _L, group, 0)
        pltpu.sync_copy(w1v, w1_hbm.at[pl.ds(base, tpw)])
        pltpu.sync_copy(w2v, w2_hbm.at[pl.ds(base, tpw)])
        pltpu.sync_copy(i1v, i1_hbm.at[pl.ds(base, tpw)])
        pltpu.sync_copy(i2v, i2_hbm.at[pl.ds(base, tpw)])

    return sc_router


def kernel(hidden_states, gate_w, gate_b):
    T, _ = hidden_states.shape
    ts = T // _NSLICE
    sc_router = _make_sc_router(ts)
    parts = []
    for s in range(_NSLICE):
        logits_t = _tc_logits_t(
            jax.lax.slice_in_dim(hidden_states, s * ts, (s + 1) * ts),
            gate_w, gate_b,
        )
        parts.append(sc_router(logits_t))
    w1, w2, i1, i2 = (jnp.concatenate([p[j] for p in parts]) for j in range(4))
    weights = jnp.stack([w1, w2], axis=-1)
    idx = jnp.stack([i1, i2], axis=-1)
    return (weights, idx)


# speedup vs baseline: 2.1709x; 2.1709x over previous
import functools
import jax
import jax.numpy as jnp
from jax import lax
from jax.experimental import pallas as pl
from jax.experimental.pallas import tpu as pltpu
from jax.experimental.pallas import tpu_sc as plsc

_TB = 2048
_NE = 64
_NC = 2
_NS = 16
_NW = _NC * _NS
_L = 16
_UNROLL = 4
_NSLICE = 1


def _logits_body(x_ref, w_ref, b_ref, out_ref):
    x = x_ref[...]
    w = w_ref[...]
    lt = jax.lax.dot_general(
        w, x, (((1,), (1,)), ((), ())), preferred_element_type=jnp.float32
    )
    out_ref[...] = lt + b_ref[...]


def _tc_logits_t(hidden_states, gate_w, gate_b):
    T, H = hidden_states.shape
    return pl.pallas_call(
        _logits_body,
        grid=(T // _TB,),
        in_specs=[
            pl.BlockSpec((_TB, H), lambda i: (i, 0)),
            pl.BlockSpec((_NE, H), lambda i: (0, 0)),
            pl.BlockSpec((_NE, 1), lambda i: (0, 0)),
        ],
        out_specs=pl.BlockSpec((_NE, _TB), lambda i: (0, i)),
        out_shape=jax.ShapeDtypeStruct((_NE, T), jnp.float32),
    )(hidden_states, gate_w, gate_b.reshape(_NE, 1))


def _make_sc_router(T):
    tpw = T // _NW
    mesh = plsc.VectorSubcoreMesh(core_axis_name="c", subcore_axis_name="s")

    @functools.partial(
        pl.kernel,
        mesh=mesh,
        out_type=[
            jax.ShapeDtypeStruct((T,), jnp.float32),
            jax.ShapeDtypeStruct((T,), jnp.float32),
            jax.ShapeDtypeStruct((T,), jnp.int32),
            jax.ShapeDtypeStruct((T,), jnp.int32),
        ],
        scratch_types=[
            pltpu.VMEM((_NE, tpw), jnp.float32),
            pltpu.VMEM((tpw,), jnp.float32),
            pltpu.VMEM((tpw,), jnp.float32),
            pltpu.VMEM((tpw,), jnp.int32),
            pltpu.VMEM((tpw,), jnp.int32),
        ],
    )
    def sc_router(logits_hbm, w1_hbm, w2_hbm, i1_hbm, i2_hbm,
                  chunk, w1v, w2v, i1v, i2v):
        wid = lax.axis_index("s") * _NC + lax.axis_index("c")
        base = wid * tpw
        pltpu.sync_copy(logits_hbm.at[:, pl.ds(base, tpw)], chunk)

        def group(g, _):
            g16 = g * _L
            neg = jnp.full((_L,), -jnp.inf, jnp.float32)
            zz = jnp.zeros((_L,), jnp.int32)

            def estep(k, c):
                for d in range(_UNROLL):
                    m1, m2, j1, j2 = c
                    e = k * _UNROLL + d
                    v = chunk[e, pl.ds(g16, _L)]
                    ev = jnp.full((_L,), e, jnp.int32)
                    gt1 = v > m1
                    gt2 = v > m2
                    m2n = jnp.where(gt1, m1, jnp.where(gt2, v, m2))
                    j2n = jnp.where(gt1, j1, jnp.where(gt2, ev, j2))
                    m1n = jnp.where(gt1, v, m1)
                    j1n = jnp.where(gt1, ev, j1)
                    c = (m1n, m2n, j1n, j2n)
                return c

            m1, m2, j1, j2 = lax.fori_loop(
                0, _NE // _UNROLL, estep, (neg, neg, zz, zz)
            )
            ex = jnp.exp(m2 - m1)
            wa = 1.0 / (1.0 + ex)
            w1v[pl.ds(g16, _L)] = wa
            w2v[pl.ds(g16, _L)] = ex * wa
            i1v[pl.ds(g16, _L)] = j1
            i2v[pl.ds(g16, _L)] = j2
            return 0

        lax.fori_loop(0, tpw // _L, group, 0)
        pltpu.sync_copy(w1v, w1_hbm.at[pl.ds(base, tpw)])
        pltpu.sync_copy(w2v, w2_hbm.at[pl.ds(base, tpw)])
        pltpu.sync_copy(i1v, i1_hbm.at[pl.ds(base, tpw)])
        pltpu.sync_copy(i2v, i2_hbm.at[pl.ds(base, tpw)])

    return sc_router


def kernel(hidden_states, gate_w, gate_b):
    T, _ = hidden_states.shape
    ts = T // _NSLICE
    sc_router = _make_sc_router(ts)
    parts = []
    for s in range(_NSLICE):
        logits_t = _tc_logits_t(
            jax.lax.slice_in_dim(hidden_states, s * ts, (s + 1) * ts),
            gate_w, gate_b,
        )
        parts.append(sc_router(logits_t))
    w1, w2, i1, i2 = (jnp.concatenate([p[j] for p in parts]) for j in range(4))
    weights = jnp.stack([w1, w2], axis=-1)
    idx = jnp.stack([i1, i2], axis=-1)
    return (weights, idx)
